# baseline (speedup 1.0000x reference)
import functools

import jax
import jax.numpy as jnp
from jax import lax
from jax.experimental import pallas as pl
from jax.experimental.pallas import tpu as pltpu
from jax.experimental.pallas import tpu_sc as plsc

_B = 16384
_F = 26
_D = 64
_DP = 128
_R = _B * _F
_N = 1000000
_NC = 2
_NS = 16
_NW = _NC * _NS
_CH = 128

_TBLK = 8192
_TGRID = 62
_H = _TBLK * _TGRID
_BBLK = 8192

_F1 = 16
_F2 = _F - _F1


def _transform_body(ta_ref, tb_ref, w2_ref, o_ref):
    lhs = jnp.concatenate([ta_ref[...], tb_ref[...]], axis=0)
    o_ref[...] = lax.dot_general(
        lhs, w2_ref[...],
        dimension_numbers=(((0,), (0,)), ((), ())),
        preferred_element_type=jnp.float32,
    )


def _make_gather_body(row_base, nch):
    def body(table, idx2d, out, idx_v, rows_a, rows_b, sem_a, sem_b):
        wid = lax.axis_index("s") * _NC + lax.axis_index("c")
        myrow = row_base + wid * nch
        out0 = wid * nch
        pltpu.sync_copy(idx2d.at[pl.ds(myrow, nch)], idx_v)

        pltpu.async_copy(table.at[idx_v.at[0]], rows_a, sem_a)

        def step(j, _):
            @pl.when(j + 1 < nch)
            def _start_next():
                pltpu.async_copy(table.at[idx_v.at[j + 1]], rows_b, sem_b)

            pltpu.make_async_copy(table.at[idx_v.at[j]], rows_a, sem_a).wait()
            pltpu.sync_copy(rows_a, out.at[pl.ds((out0 + j) * _CH, _CH)])

            @pl.when(j + 2 < nch)
            def _start_next2():
                pltpu.async_copy(table.at[idx_v.at[j + 2]], rows_a, sem_a)

            @pl.when(j + 1 < nch)
            def _drain_b():
                pltpu.make_async_copy(table.at[idx_v.at[j + 1]], rows_b,
                                      sem_b).wait()
                pltpu.sync_copy(rows_b, out.at[pl.ds((out0 + j + 1) * _CH, _CH)])

            return 0

        lax.fori_loop(0, nch // 2, lambda j, c: step(2 * j, c), 0,
                      unroll=False)

    return body


def _trwrite_body(sel_ref, g_ref, *rest):
    o_ref = rest[-1]
    f = pl.program_id(1)
    t = g_ref[0].T
    s = sel_ref[pl.ds(f, 1), :]
    o_ref[0] = jnp.where(s > 0, t[_D:, :], t[:_D, :])


def _gather_half(transformed, idx2d, row_base, nfields):
    nch = nfields * _B // _NW // _CH
    return pl.kernel(
        _make_gather_body(row_base, nch),
        out_type=jax.ShapeDtypeStruct((nfields * _B, _DP), jnp.float32),
        mesh=plsc.VectorSubcoreMesh(core_axis_name="c", subcore_axis_name="s"),
        scratch_types=[
            pltpu.VMEM((nch, _CH), jnp.int32),
            pltpu.VMEM((_CH, _DP), jnp.float32),
            pltpu.VMEM((_CH, _DP), jnp.float32),
            pltpu.SemaphoreType.DMA,
            pltpu.SemaphoreType.DMA,
        ],
    )(transformed, idx2d)


def kernel(x, emb_table, W):
    xt = x.T.astype(jnp.int32)
    sel = (xt >= _H).astype(jnp.int32)
    idx2d = (xt - _H * (xt >= _H)).reshape(_R // _CH, _CH)
    table_t = emb_table.T
    wt = W.T
    w2 = (jnp.zeros((_DP, _DP), jnp.float32)
          .at[:_D, :_D].set(wt).at[_D:, _D:].set(wt))

    transformed = pl.pallas_call(
        _transform_body,
        name='transform',
        grid=(_TGRID,),
        in_specs=[
            pl.BlockSpec((_D, _TBLK), lambda i: (0, i)),
            pl.BlockSpec((_D, _TBLK),
                         lambda i: (0, jnp.minimum(i + _TGRID, _N // _TBLK))),
            pl.BlockSpec((_DP, _DP), lambda i: (0, 0)),
        ],
        out_specs=pl.BlockSpec((_TBLK, _DP), lambda i: (i, 0)),
        out_shape=jax.ShapeDtypeStruct((_H, _DP), jnp.float32),
    )(table_t, table_t, w2)

    g_lo = _gather_half(transformed, idx2d, 0, _F1)
    transformed2, g_lo = lax.optimization_barrier((transformed, g_lo))
    g_hi = _gather_half(transformed2, idx2d, _F1 * _B // _CH, _F2)

    out_shape = jax.ShapeDtypeStruct((_F, _D, _B), jnp.float32)
    sel_lo, sel_hi = sel[:_F1], sel[_F1:]
    o1 = pl.pallas_call(
        _trwrite_body,
        name='trwrite_lo',
        grid=(_B // _BBLK, _F1),
        in_specs=[
            pl.BlockSpec((_F1, _BBLK), lambda j, f: (0, j)),
            pl.BlockSpec((1, _BBLK, _DP), lambda j, f: (f, j, 0)),
        ],
        out_specs=pl.BlockSpec((1, _D, _BBLK), lambda j, f: (f, 0, j)),
        out_shape=out_shape,
    )(sel_lo, g_lo.reshape(_F1, _B, _DP))

    out_t = pl.pallas_call(
        _trwrite_body,
        name='trwrite_hi',
        grid=(_B // _BBLK, _F2),
        in_specs=[
            pl.BlockSpec((_F2, _BBLK), lambda j, f: (0, j)),
            pl.BlockSpec((1, _BBLK, _DP), lambda j, f: (f, j, 0)),
            pl.BlockSpec(memory_space=pl.ANY),
        ],
        out_specs=pl.BlockSpec((1, _D, _BBLK), lambda j, f: (f + _F1, 0, j)),
        out_shape=out_shape,
        input_output_aliases={2: 0},
    )(sel_hi, g_hi.reshape(_F2, _B, _DP), o1)

    return out_t.transpose(2, 0, 1)


# speedup vs baseline: 1.0098x; 1.0098x over previous
import functools

import jax
import jax.numpy as jnp
from jax import lax
from jax.experimental import pallas as pl
from jax.experimental.pallas import tpu as pltpu
from jax.experimental.pallas import tpu_sc as plsc

_B = 16384
_F = 26
_D = 64
_DP = 128
_R = _B * _F
_N = 1000000
_NC = 2
_NS = 16
_NW = _NC * _NS
_CH = 128

_TBLK = 16384
_TGRID = 31
_H = _TBLK * _TGRID
_BBLK = 8192

_F1 = 16
_F2 = _F - _F1


def _transform_body(ta_ref, tb_ref, w2_ref, o_ref):
    lhs = jnp.concatenate([ta_ref[...], tb_ref[...]], axis=0)
    o_ref[...] = lax.dot_general(
        lhs, w2_ref[...],
        dimension_numbers=(((0,), (0,)), ((), ())),
        preferred_element_type=jnp.float32,
    )


def _make_gather_body(row_base, nch):
    def body(table, idx2d, out, idx_v, rows_a, rows_b, sem_a, sem_b):
        wid = lax.axis_index("s") * _NC + lax.axis_index("c")
        myrow = row_base + wid * nch
        out0 = wid * nch
        pltpu.sync_copy(idx2d.at[pl.ds(myrow, nch)], idx_v)

        pltpu.async_copy(table.at[idx_v.at[0]], rows_a, sem_a)

        def step(j, _):
            @pl.when(j + 1 < nch)
            def _start_next():
                pltpu.async_copy(table.at[idx_v.at[j + 1]], rows_b, sem_b)

            pltpu.make_async_copy(table.at[idx_v.at[j]], rows_a, sem_a).wait()
            pltpu.sync_copy(rows_a, out.at[pl.ds((out0 + j) * _CH, _CH)])

            @pl.when(j + 2 < nch)
            def _start_next2():
                pltpu.async_copy(table.at[idx_v.at[j + 2]], rows_a, sem_a)

            @pl.when(j + 1 < nch)
            def _drain_b():
                pltpu.make_async_copy(table.at[idx_v.at[j + 1]], rows_b,
                                      sem_b).wait()
                pltpu.sync_copy(rows_b, out.at[pl.ds((out0 + j + 1) * _CH, _CH)])

            return 0

        lax.fori_loop(0, nch // 2, lambda j, c: step(2 * j, c), 0,
                      unroll=False)

    return body


def _trwrite_body(sel_ref, g_ref, *rest):
    o_ref = rest[-1]
    f = pl.program_id(1)
    t = g_ref[0].T
    s = sel_ref[pl.ds(f, 1), :]
    o_ref[0] = jnp.where(s > 0, t[_D:, :], t[:_D, :])


def _gather_half(transformed, idx2d, row_base, nfields):
    nch = nfields * _B // _NW // _CH
    return pl.kernel(
        _make_gather_body(row_base, nch),
        out_type=jax.ShapeDtypeStruct((nfields * _B, _DP), jnp.float32),
        mesh=plsc.VectorSubcoreMesh(core_axis_name="c", subcore_axis_name="s"),
        scratch_types=[
            pltpu.VMEM((nch, _CH), jnp.int32),
            pltpu.VMEM((_CH, _DP), jnp.float32),
            pltpu.VMEM((_CH, _DP), jnp.float32),
            pltpu.SemaphoreType.DMA,
            pltpu.SemaphoreType.DMA,
        ],
    )(transformed, idx2d)


def kernel(x, emb_table, W):
    xt = x.T.astype(jnp.int32)
    sel = (xt >= _H).astype(jnp.int32)
    idx2d = (xt - _H * (xt >= _H)).reshape(_R // _CH, _CH)
    table_t = emb_table.T
    wt = W.T
    w2 = (jnp.zeros((_DP, _DP), jnp.float32)
          .at[:_D, :_D].set(wt).at[_D:, _D:].set(wt))

    transformed = pl.pallas_call(
        _transform_body,
        name='transform',
        grid=(_TGRID,),
        in_specs=[
            pl.BlockSpec((_D, _TBLK), lambda i: (0, i)),
            pl.BlockSpec((_D, _TBLK),
                         lambda i: (0, jnp.minimum(i + _TGRID, _N // _TBLK))),
            pl.BlockSpec((_DP, _DP), lambda i: (0, 0)),
        ],
        out_specs=pl.BlockSpec((_TBLK, _DP), lambda i: (i, 0)),
        out_shape=jax.ShapeDtypeStruct((_H, _DP), jnp.float32),
    )(table_t, table_t, w2)

    g_lo = _gather_half(transformed, idx2d, 0, _F1)
    transformed2, g_lo = lax.optimization_barrier((transformed, g_lo))
    g_hi = _gather_half(transformed2, idx2d, _F1 * _B // _CH, _F2)

    out_shape = jax.ShapeDtypeStruct((_F, _D, _B), jnp.float32)
    sel_lo, sel_hi = sel[:_F1], sel[_F1:]
    o1 = pl.pallas_call(
        _trwrite_body,
        name='trwrite_lo',
        grid=(_B // _BBLK, _F1),
        in_specs=[
            pl.BlockSpec((_F1, _BBLK), lambda j, f: (0, j)),
            pl.BlockSpec((1, _BBLK, _DP), lambda j, f: (f, j, 0)),
        ],
        out_specs=pl.BlockSpec((1, _D, _BBLK), lambda j, f: (f, 0, j)),
        out_shape=out_shape,
    )(sel_lo, g_lo.reshape(_F1, _B, _DP))

    out_t = pl.pallas_call(
        _trwrite_body,
        name='trwrite_hi',
        grid=(_B // _BBLK, _F2),
        in_specs=[
            pl.BlockSpec((_F2, _BBLK), lambda j, f: (0, j)),
            pl.BlockSpec((1, _BBLK, _DP), lambda j, f: (f, j, 0)),
            pl.BlockSpec(memory_space=pl.ANY),
        ],
        out_specs=pl.BlockSpec((1, _D, _BBLK), lambda j, f: (f + _F1, 0, j)),
        out_shape=out_shape,
        input_output_aliases={2: 0},
    )(sel_hi, g_hi.reshape(_F2, _B, _DP), o1)

    return out_t.transpose(2, 0, 1)
